# Initial kernel scaffold; baseline (speedup 1.0000x reference)
import jax
import jax.numpy as jnp
from jax import lax
from jax.experimental import pallas as pl
from jax.experimental.pallas import tpu as pltpu

NQ = 8
K = 1024
CD = 256
ID = 512


def _dot(a, b, dims):
    return lax.dot_general(
        a, b, (dims, ((), ())),
        precision=lax.Precision.HIGHEST,
        preferred_element_type=jnp.float32,
    )


def _rvq_block(z_ref, V_in_ref, G_in_ref, B_in_ref, V_out_ref, G_out_ref,
               B_out_ref, CB_ref, qout_ref, idx_ref, aq_ref):
    res = z_ref[0]
    qout = jnp.zeros_like(res)
    row_iota = lax.broadcasted_iota(jnp.int32, (K, res.shape[1]), 0)

    for i in range(NQ):
        v_in = V_in_ref[i]
        n_in = jnp.sqrt(jnp.sum(v_in * v_in, axis=1, keepdims=True))
        w_in = G_in_ref[:, i:i + 1] * v_in / n_in
        z_e = _dot(w_in, res, ((1,), (0,))) + B_in_ref[:, i:i + 1]

        cb = CB_ref[i]
        n_cb = jnp.sqrt(jnp.sum(cb * cb, axis=1, keepdims=True))
        cb_n = cb / jnp.maximum(n_cb, 1e-12)
        s2 = jnp.sum(cb_n * cb_n, axis=1, keepdims=True)
        n_e = jnp.sqrt(jnp.sum(z_e * z_e, axis=0, keepdims=True))
        enc_n = z_e / jnp.maximum(n_e, 1e-12)
        s1 = jnp.sum(enc_n * enc_n, axis=0, keepdims=True)
        dots = _dot(cb_n, enc_n, ((1,), (0,)))
        score = -((s1 - 2.0 * dots) + s2)

        m = jnp.max(score, axis=0, keepdims=True)
        idx = jnp.min(jnp.where(score == m, row_iota, K),
                      axis=0, keepdims=True)

        oh = (row_iota == idx).astype(jnp.float32)
        zq_cd = _dot(cb, oh, ((0,), (0,)))
        zq_st = z_e + (zq_cd - z_e)

        v_out = V_out_ref[i]
        n_out = jnp.sqrt(jnp.sum(v_out * v_out, axis=1, keepdims=True))
        w_out = G_out_ref[:, i:i + 1] * v_out / n_out
        zq_out = _dot(w_out, zq_st, ((1,), (0,))) + B_out_ref[:, i:i + 1]

        qout = qout + zq_out
        res = res - zq_out
        aq_ref[i, 0] = zq_out
        idx_ref[i, 0] = idx

    qout_ref[0] = qout


def kernel(z, V_in, G_in, B_in, V_out, G_out, B_out, CB):
    B, D, T = z.shape
    full = lambda *shape: pl.BlockSpec(shape, lambda b: (0,) * len(shape))
    out_shapes = (
        jax.ShapeDtypeStruct((B, D, T), jnp.float32),
        jax.ShapeDtypeStruct((NQ, B, 1, T), jnp.int32),
        jax.ShapeDtypeStruct((NQ, B, D, T), jnp.float32),
    )
    qout, idx4, aq = pl.pallas_call(
        _rvq_block,
        grid=(B,),
        in_specs=[
            pl.BlockSpec((1, D, T), lambda b: (b, 0, 0)),
            full(NQ, CD, D),
            full(CD, NQ),
            full(CD, NQ),
            full(NQ, D, CD),
            full(D, NQ),
            full(D, NQ),
            full(NQ, K, CD),
        ],
        out_specs=(
            pl.BlockSpec((1, D, T), lambda b: (b, 0, 0)),
            pl.BlockSpec((NQ, 1, 1, T), lambda b: (0, b, 0, 0)),
            pl.BlockSpec((NQ, 1, D, T), lambda b: (0, b, 0, 0)),
        ),
        out_shape=out_shapes,
        compiler_params=pltpu.CompilerParams(
            dimension_semantics=("arbitrary",),
            vmem_limit_bytes=100 * 1024 * 1024,
        ),
    )(z, V_in, G_in.T, B_in.T, V_out, G_out.T, B_out.T, CB)

    zeros = jnp.zeros((NQ,), dtype=jnp.float32)
    return (qout, idx4.reshape(NQ, B, T), zeros, zeros, aq)


# speedup vs baseline: 1.7144x; 1.7144x over previous
import jax
import jax.numpy as jnp
from jax import lax
from jax.experimental import pallas as pl
from jax.experimental.pallas import tpu as pltpu

NQ = 8
K = 1024
CD = 256
ID = 512


def _dot(a, b, dims, precision=lax.Precision.DEFAULT):
    return lax.dot_general(
        a, b, (dims, ((), ())),
        precision=precision,
        preferred_element_type=jnp.float32,
    )


def _rvq_block(z_ref, W_in_ref, B_in_ref, W_out_ref, B_out_ref, CB_ref,
               CBn_ref, qout_ref, idx_ref, aq_ref):
    res = z_ref[0]
    qout = jnp.zeros_like(res)
    row_iota = lax.broadcasted_iota(jnp.int32, (K, res.shape[1]), 0)

    for i in range(NQ):
        z_e = _dot(W_in_ref[i], res, ((1,), (0,))) + B_in_ref[:, i:i + 1]

        cb_n = CBn_ref[i]
        s2 = jnp.sum(cb_n * cb_n, axis=1, keepdims=True)
        n_e = jnp.sqrt(jnp.sum(z_e * z_e, axis=0, keepdims=True))
        enc_n = z_e / jnp.maximum(n_e, 1e-12)
        s1 = jnp.sum(enc_n * enc_n, axis=0, keepdims=True)
        dots = _dot(cb_n, enc_n, ((1,), (0,)))
        score = -((s1 - 2.0 * dots) + s2)

        m = jnp.max(score, axis=0, keepdims=True)
        idx = jnp.min(jnp.where(score == m, row_iota, K),
                      axis=0, keepdims=True)

        oh = (row_iota == idx).astype(jnp.float32)
        zq_cd = _dot(CB_ref[i], oh, ((0,), (0,)),
                     precision=lax.Precision.HIGHEST)
        zq_st = z_e + (zq_cd - z_e)

        zq_out = _dot(W_out_ref[i], zq_st, ((1,), (0,))) + B_out_ref[:, i:i + 1]

        qout = qout + zq_out
        res = res - zq_out
        aq_ref[i, 0] = zq_out
        idx_ref[i, 0] = idx

    qout_ref[0] = qout


def kernel(z, V_in, G_in, B_in, V_out, G_out, B_out, CB):
    B, D, T = z.shape
    W_in = jnp.stack([
        G_in[i][:, None] * V_in[i] / jnp.linalg.norm(V_in[i], axis=1, keepdims=True)
        for i in range(NQ)])
    W_out = jnp.stack([
        G_out[i][:, None] * V_out[i] / jnp.linalg.norm(V_out[i], axis=1, keepdims=True)
        for i in range(NQ)])
    CB_n = jnp.stack([
        CB[i] / jnp.maximum(jnp.linalg.norm(CB[i], axis=1, keepdims=True), 1e-12)
        for i in range(NQ)])

    full = lambda *shape: pl.BlockSpec(shape, lambda b: (0,) * len(shape))
    out_shapes = (
        jax.ShapeDtypeStruct((B, D, T), jnp.float32),
        jax.ShapeDtypeStruct((NQ, B, 1, T), jnp.int32),
        jax.ShapeDtypeStruct((NQ, B, D, T), jnp.float32),
    )
    qout, idx4, aq = pl.pallas_call(
        _rvq_block,
        grid=(B,),
        in_specs=[
            pl.BlockSpec((1, D, T), lambda b: (b, 0, 0)),
            full(NQ, CD, D),
            full(CD, NQ),
            full(NQ, D, CD),
            full(D, NQ),
            full(NQ, K, CD),
            full(NQ, K, CD),
        ],
        out_specs=(
            pl.BlockSpec((1, D, T), lambda b: (b, 0, 0)),
            pl.BlockSpec((NQ, 1, 1, T), lambda b: (0, b, 0, 0)),
            pl.BlockSpec((NQ, 1, D, T), lambda b: (0, b, 0, 0)),
        ),
        out_shape=out_shapes,
        compiler_params=pltpu.CompilerParams(
            dimension_semantics=("arbitrary",),
            vmem_limit_bytes=100 * 1024 * 1024,
        ),
    )(z, W_in, B_in.T, W_out, B_out.T, CB, CB_n)

    zeros = jnp.zeros((NQ,), dtype=jnp.float32)
    return (qout, idx4.reshape(NQ, B, T), zeros, zeros, aq)


# speedup vs baseline: 1.7204x; 1.0035x over previous
import jax
import jax.numpy as jnp
from jax import lax
from jax.experimental import pallas as pl
from jax.experimental.pallas import tpu as pltpu

NQ = 8
K = 1024
CD = 256
ID = 512


def _dot(a, b, dims, precision=lax.Precision.DEFAULT):
    return lax.dot_general(
        a, b, (dims, ((), ())),
        precision=precision,
        preferred_element_type=jnp.float32,
    )


def _rvq_block(z_ref, W_in_ref, B_in_ref, W_out_ref, B_out_ref, CB_ref,
               CBn_ref, qout_ref, idx_ref, aq_ref):
    res = z_ref[0]
    qout = jnp.zeros_like(res)
    row_iota = lax.broadcasted_iota(jnp.int32, (K, res.shape[1]), 0)

    for i in range(NQ):
        z_e = _dot(W_in_ref[i], res, ((1,), (0,))) + B_in_ref[:, i:i + 1]

        cb_n = CBn_ref[i]
        s2 = jnp.sum(cb_n * cb_n, axis=1, keepdims=True)
        n_e = jnp.sqrt(jnp.sum(z_e * z_e, axis=0, keepdims=True))
        enc_n = z_e / jnp.maximum(n_e, 1e-12)
        s1 = jnp.sum(enc_n * enc_n, axis=0, keepdims=True)
        dots = _dot(cb_n, enc_n, ((1,), (0,)))
        score = -((s1 - 2.0 * dots) + s2)

        m = jnp.max(score, axis=0, keepdims=True)
        idx = jnp.min(jnp.where(score == m, row_iota, K),
                      axis=0, keepdims=True)

        oh = (row_iota == idx).astype(jnp.float32)
        zq_cd = _dot(CB_ref[i], oh, ((0,), (0,)),
                     precision=lax.Precision.HIGHEST)
        zq_st = z_e + (zq_cd - z_e)

        zq_out = _dot(W_out_ref[i], zq_st, ((1,), (0,))) + B_out_ref[:, i:i + 1]

        qout = qout + zq_out
        res = res - zq_out
        aq_ref[i, 0] = zq_out
        idx_ref[i, 0] = idx

    qout_ref[0] = qout


def kernel(z, V_in, G_in, B_in, V_out, G_out, B_out, CB):
    B, D, T = z.shape
    W_in = jnp.stack([
        G_in[i][:, None] * V_in[i] / jnp.linalg.norm(V_in[i], axis=1, keepdims=True)
        for i in range(NQ)])
    W_out = jnp.stack([
        G_out[i][:, None] * V_out[i] / jnp.linalg.norm(V_out[i], axis=1, keepdims=True)
        for i in range(NQ)])
    CB_n = jnp.stack([
        CB[i] / jnp.maximum(jnp.linalg.norm(CB[i], axis=1, keepdims=True), 1e-12)
        for i in range(NQ)])

    full = lambda *shape: pl.BlockSpec(shape, lambda b: (0,) * len(shape))
    out_shapes = (
        jax.ShapeDtypeStruct((B, D, T), jnp.float32),
        jax.ShapeDtypeStruct((NQ, B, 1, T), jnp.int32),
        jax.ShapeDtypeStruct((NQ, B, D, T), jnp.float32),
    )
    qout, idx4, aq = pl.pallas_call(
        _rvq_block,
        grid=(B,),
        in_specs=[
            pl.BlockSpec((1, D, T), lambda b: (b, 0, 0)),
            full(NQ, CD, D),
            full(CD, NQ),
            full(NQ, D, CD),
            full(D, NQ),
            full(NQ, K, CD),
            full(NQ, K, CD),
        ],
        out_specs=(
            pl.BlockSpec((1, D, T), lambda b: (b, 0, 0)),
            pl.BlockSpec((NQ, 1, 1, T), lambda b: (0, b, 0, 0)),
            pl.BlockSpec((NQ, 1, D, T), lambda b: (0, b, 0, 0)),
        ),
        out_shape=out_shapes,
        compiler_params=pltpu.CompilerParams(
            dimension_semantics=("parallel",),
            vmem_limit_bytes=100 * 1024 * 1024,
        ),
    )(z, W_in, B_in.T, W_out, B_out.T, CB, CB_n)

    zeros = jnp.zeros((NQ,), dtype=jnp.float32)
    return (qout, idx4.reshape(NQ, B, T), zeros, zeros, aq)


# speedup vs baseline: 2.3568x; 1.3699x over previous
import jax
import jax.numpy as jnp
from jax import lax
from jax.experimental import pallas as pl
from jax.experimental.pallas import tpu as pltpu

NQ = 8
K = 1024
CD = 256
ID = 512


def _dot(a, b, dims, precision=lax.Precision.DEFAULT):
    return lax.dot_general(
        a, b, (dims, ((), ())),
        precision=precision,
        preferred_element_type=jnp.float32,
    )


def _rvq_block(z_ref, W_in_ref, B_in_ref, W_out_ref, B_out_ref, CBhi_ref,
               CBmid_ref, CBlo_ref, CBn_ref, qout_ref, idx_ref, aq_ref):
    res = z_ref[0]
    qout = jnp.zeros_like(res)
    row_iota = lax.broadcasted_iota(jnp.int32, (K, res.shape[1]), 0)

    for i in range(NQ):
        z_e = _dot(W_in_ref[i], res, ((1,), (0,))) + B_in_ref[:, i:i + 1]

        cb_n = CBn_ref[i]
        s2 = jnp.sum(cb_n * cb_n, axis=1, keepdims=True)
        n_e = jnp.sqrt(jnp.sum(z_e * z_e, axis=0, keepdims=True))
        enc_n = z_e / jnp.maximum(n_e, 1e-12)
        s1 = jnp.sum(enc_n * enc_n, axis=0, keepdims=True)
        dots = _dot(cb_n, enc_n, ((1,), (0,)))
        score = -((s1 - 2.0 * dots) + s2)

        m = jnp.max(score, axis=0, keepdims=True)
        idx = jnp.min(jnp.where(score == m, row_iota, K),
                      axis=0, keepdims=True)

        oh = (row_iota == idx).astype(jnp.bfloat16)
        zq_cd = ((_dot(CBhi_ref[i], oh, ((0,), (0,)))
                  + _dot(CBmid_ref[i], oh, ((0,), (0,))))
                 + _dot(CBlo_ref[i], oh, ((0,), (0,))))
        zq_st = z_e + (zq_cd - z_e)

        zq_out = _dot(W_out_ref[i], zq_st, ((1,), (0,))) + B_out_ref[:, i:i + 1]

        qout = qout + zq_out
        res = res - zq_out
        aq_ref[i, 0] = zq_out
        idx_ref[i, 0] = idx

    qout_ref[0] = qout


def kernel(z, V_in, G_in, B_in, V_out, G_out, B_out, CB):
    B, D, T = z.shape
    W_in = jnp.stack([
        G_in[i][:, None] * V_in[i] / jnp.linalg.norm(V_in[i], axis=1, keepdims=True)
        for i in range(NQ)])
    W_out = jnp.stack([
        G_out[i][:, None] * V_out[i] / jnp.linalg.norm(V_out[i], axis=1, keepdims=True)
        for i in range(NQ)])
    CB_n = jnp.stack([
        CB[i] / jnp.maximum(jnp.linalg.norm(CB[i], axis=1, keepdims=True), 1e-12)
        for i in range(NQ)])
    CB_hi = CB.astype(jnp.bfloat16)
    r1 = CB - CB_hi.astype(jnp.float32)
    CB_mid = r1.astype(jnp.bfloat16)
    CB_lo = (r1 - CB_mid.astype(jnp.float32)).astype(jnp.bfloat16)

    full = lambda *shape: pl.BlockSpec(shape, lambda b: (0,) * len(shape))
    out_shapes = (
        jax.ShapeDtypeStruct((B, D, T), jnp.float32),
        jax.ShapeDtypeStruct((NQ, B, 1, T), jnp.int32),
        jax.ShapeDtypeStruct((NQ, B, D, T), jnp.float32),
    )
    qout, idx4, aq = pl.pallas_call(
        _rvq_block,
        grid=(B,),
        in_specs=[
            pl.BlockSpec((1, D, T), lambda b: (b, 0, 0)),
            full(NQ, CD, D),
            full(CD, NQ),
            full(NQ, D, CD),
            full(D, NQ),
            full(NQ, K, CD),
            full(NQ, K, CD),
            full(NQ, K, CD),
            full(NQ, K, CD),
        ],
        out_specs=(
            pl.BlockSpec((1, D, T), lambda b: (b, 0, 0)),
            pl.BlockSpec((NQ, 1, 1, T), lambda b: (0, b, 0, 0)),
            pl.BlockSpec((NQ, 1, D, T), lambda b: (0, b, 0, 0)),
        ),
        out_shape=out_shapes,
        compiler_params=pltpu.CompilerParams(
            dimension_semantics=("parallel",),
            vmem_limit_bytes=100 * 1024 * 1024,
        ),
    )(z, W_in, B_in.T, W_out, B_out.T, CB_hi, CB_mid, CB_lo, CB_n)

    zeros = jnp.zeros((NQ,), dtype=jnp.float32)
    return (qout, idx4.reshape(NQ, B, T), zeros, zeros, aq)
